# baseline (speedup 1.0000x reference)
import functools

import jax
import jax.numpy as jnp
from jax.experimental import pallas as pl
from jax.experimental.pallas import tpu as pltpu

_N = 8192
_D = 2048
_H = 2048
_E = 8
_NDIFF = 3
_TM = 256


def _gating_body(x_ref, lab_ref, emb_ref, g1_ref, g2_ref, b_ref, idx_ref, val_ref):
    x = x_ref[...]
    logits = jnp.dot(x, g1_ref[...], preferred_element_type=jnp.float32)
    pre = jnp.dot(emb_ref[...], g2_ref[...], preferred_element_type=jnp.float32)
    lab = lab_ref[...]
    iota8 = jax.lax.broadcasted_iota(jnp.int32, (_TM, _E), 1)
    acc = jnp.zeros((_TM, _E), jnp.float32)
    for l in range(_NDIFF):
        acc = acc + jnp.where(lab == l, pre[l][None, :], 0.0)
    logits = logits + acc + b_ref[...]
    m = jnp.max(logits, axis=1, keepdims=True)
    ex = jnp.exp(logits - m)
    p = ex / jnp.sum(ex, axis=1, keepdims=True)
    m0 = jnp.max(p, axis=1, keepdims=True)
    i0 = jnp.min(jnp.where(p == m0, iota8, _E), axis=1, keepdims=True)
    sel0 = iota8 == i0
    p1 = jnp.where(sel0, -jnp.inf, p)
    m1 = jnp.max(p1, axis=1, keepdims=True)
    i1 = jnp.min(jnp.where(p1 == m1, iota8, _E), axis=1, keepdims=True)
    idx_ref[...] = jnp.where(iota8 == 0, i0, jnp.where(iota8 == 1, i1, 0))
    val_ref[...] = jnp.where(iota8 == 0, m0, jnp.where(iota8 == 1, m1, 0.0))


def _gating(x, labels, emb_pad, g1, g2, b2):
    n_tiles = _N // _TM
    return pl.pallas_call(
        _gating_body,
        grid=(n_tiles,),
        in_specs=[
            pl.BlockSpec((_TM, _D), lambda j: (j, 0)),
            pl.BlockSpec((_TM, 1), lambda j: (j, 0)),
            pl.BlockSpec((_E, _D), lambda j: (0, 0)),
            pl.BlockSpec((_D, _E), lambda j: (0, 0)),
            pl.BlockSpec((_D, _E), lambda j: (0, 0)),
            pl.BlockSpec((1, _E), lambda j: (0, 0)),
        ],
        out_specs=[
            pl.BlockSpec((_TM, _E), lambda j: (j, 0)),
            pl.BlockSpec((_TM, _E), lambda j: (j, 0)),
        ],
        out_shape=[
            jax.ShapeDtypeStruct((_N, _E), jnp.int32),
            jax.ShapeDtypeStruct((_N, _E), jnp.float32),
        ],
    )(x, labels, emb_pad, g1, g2, b2)


def _dense_body(x_ref, w_ref, b_ref, idx_ref, val_ref, y_ref):
    e = pl.program_id(1)
    idx = idx_ref[...]
    val = val_ref[...]
    wt = (jnp.where(idx[:, 0:1] == e, val[:, 0:1], 0.0)
          + jnp.where(idx[:, 1:2] == e, val[:, 1:2], 0.0))
    contrib = jnp.dot(x_ref[...], w_ref[0], preferred_element_type=jnp.float32)
    contrib = (contrib + b_ref[0]) * wt

    @pl.when(e == 0)
    def _():
        y_ref[...] = contrib

    @pl.when(e != 0)
    def _():
        y_ref[...] = y_ref[...] + contrib


def _dense_moe(x, expert_W, expert_b, idx8, val8):
    n_tiles = _N // _TM
    return pl.pallas_call(
        _dense_body,
        grid=(n_tiles, _E),
        in_specs=[
            pl.BlockSpec((_TM, _D), lambda j, e: (j, 0)),
            pl.BlockSpec((1, _D, _H), lambda j, e: (e, 0, 0)),
            pl.BlockSpec((1, 1, _H), lambda j, e: (e, 0, 0)),
            pl.BlockSpec((_TM, _E), lambda j, e: (j, 0)),
            pl.BlockSpec((_TM, _E), lambda j, e: (j, 0)),
        ],
        out_specs=pl.BlockSpec((_TM, _H), lambda j, e: (j, 0)),
        out_shape=jax.ShapeDtypeStruct((_N, _H), jnp.float32),
    )(x, expert_W, expert_b.reshape(_E, 1, _H), idx8, val8)


def kernel(x, difficulty_labels, emb_table, gate_W, gate_b, expert_W, expert_b):
    lab = difficulty_labels.astype(jnp.int32).reshape(_N, 1)
    emb_pad = jnp.pad(emb_table, ((0, _E - _NDIFF), (0, 0)))
    g1 = gate_W[:_D]
    g2 = gate_W[_D:]
    b2 = gate_b.reshape(1, _E)
    idx8, val8 = _gating(x, lab, emb_pad, g1, g2, b2)
    y = _dense_moe(x, expert_W, expert_b, idx8, val8)
    return y, idx8[:, :2]


# speedup vs baseline: 1.4079x; 1.4079x over previous
import functools

import jax
import jax.numpy as jnp
from jax.experimental import pallas as pl
from jax.experimental.pallas import tpu as pltpu

_N = 8192
_D = 2048
_H = 2048
_E = 8
_NDIFF = 3
_TM = 256


def _gating_body(x_ref, lab_ref, emb_ref, g1_ref, g2_ref, b_ref, idx_ref, val_ref):
    x = x_ref[...]
    logits = jnp.dot(x, g1_ref[...], preferred_element_type=jnp.float32)
    pre = jnp.dot(emb_ref[...], g2_ref[...], preferred_element_type=jnp.float32)
    lab = lab_ref[...]
    iota8 = jax.lax.broadcasted_iota(jnp.int32, (_TM, _E), 1)
    acc = jnp.zeros((_TM, _E), jnp.float32)
    for l in range(_NDIFF):
        acc = acc + jnp.where(lab == l, pre[l][None, :], 0.0)
    logits = logits + acc + b_ref[...]
    m = jnp.max(logits, axis=1, keepdims=True)
    ex = jnp.exp(logits - m)
    p = ex / jnp.sum(ex, axis=1, keepdims=True)
    m0 = jnp.max(p, axis=1, keepdims=True)
    i0 = jnp.min(jnp.where(p == m0, iota8, _E), axis=1, keepdims=True)
    sel0 = iota8 == i0
    p1 = jnp.where(sel0, -jnp.inf, p)
    m1 = jnp.max(p1, axis=1, keepdims=True)
    i1 = jnp.min(jnp.where(p1 == m1, iota8, _E), axis=1, keepdims=True)
    idx_ref[...] = jnp.where(iota8 == 0, i0, jnp.where(iota8 == 1, i1, 0))
    val_ref[...] = jnp.where(iota8 == 0, m0, jnp.where(iota8 == 1, m1, 0.0))


def _gating(x, labels, emb_pad, g1, g2, b2):
    n_tiles = _N // _TM
    return pl.pallas_call(
        _gating_body,
        grid=(n_tiles,),
        in_specs=[
            pl.BlockSpec((_TM, _D), lambda j: (j, 0)),
            pl.BlockSpec((_TM, 1), lambda j: (j, 0)),
            pl.BlockSpec((_E, _D), lambda j: (0, 0)),
            pl.BlockSpec((_D, _E), lambda j: (0, 0)),
            pl.BlockSpec((_D, _E), lambda j: (0, 0)),
            pl.BlockSpec((1, _E), lambda j: (0, 0)),
        ],
        out_specs=[
            pl.BlockSpec((_TM, _E), lambda j: (j, 0)),
            pl.BlockSpec((_TM, _E), lambda j: (j, 0)),
        ],
        out_shape=[
            jax.ShapeDtypeStruct((_N, _E), jnp.int32),
            jax.ShapeDtypeStruct((_N, _E), jnp.float32),
        ],
    )(x, labels, emb_pad, g1, g2, b2)


def _dense_body(x_ref, w_ref, b_ref, idx_ref, val_ref, y_ref):
    e = pl.program_id(1)
    idx = idx_ref[...]
    val = val_ref[...]
    wt = (jnp.where(idx[:, 0:1] == e, val[:, 0:1], 0.0)
          + jnp.where(idx[:, 1:2] == e, val[:, 1:2], 0.0))
    contrib = jnp.dot(x_ref[...], w_ref[0], preferred_element_type=jnp.float32)
    contrib = (contrib + b_ref[0]) * wt

    @pl.when(e == 0)
    def _():
        y_ref[...] = contrib

    @pl.when(e != 0)
    def _():
        y_ref[...] = y_ref[...] + contrib


def _dense_moe(x, expert_W, expert_b, idx8, val8):
    n_tiles = _N // _TM
    return pl.pallas_call(
        _dense_body,
        grid=(n_tiles, _E),
        in_specs=[
            pl.BlockSpec((_TM, _D), lambda j, e: (j, 0)),
            pl.BlockSpec((1, _D, _H), lambda j, e: (e, 0, 0)),
            pl.BlockSpec((1, 1, _H), lambda j, e: (e, 0, 0)),
            pl.BlockSpec((_TM, _E), lambda j, e: (j, 0)),
            pl.BlockSpec((_TM, _E), lambda j, e: (j, 0)),
        ],
        out_specs=pl.BlockSpec((_TM, _H), lambda j, e: (j, 0)),
        out_shape=jax.ShapeDtypeStruct((_N, _H), jnp.float32),
    )(x, expert_W, expert_b.reshape(_E, 1, _H), idx8, val8)


def kernel(x, difficulty_labels, emb_table, gate_W, gate_b, expert_W, expert_b):
    lab = difficulty_labels.astype(jnp.int32).reshape(_N, 1)
    emb_pad = jnp.pad(emb_table, ((0, _E - _NDIFF), (0, 0)))
    g1 = gate_W[:_D]
    g2 = gate_W[_D:]
    b2 = gate_b.reshape(1, _E)
    idx8, val8 = _gating(x, lab, emb_pad, g1, g2, b2)
    y = _dense_moe(x.astype(jnp.bfloat16), expert_W.astype(jnp.bfloat16),
                   expert_b, idx8, val8)
    return y, idx8[:, :2]


# speedup vs baseline: 1.5056x; 1.0694x over previous
import functools

import jax
import jax.numpy as jnp
from jax import lax
from jax.experimental import pallas as pl
from jax.experimental.pallas import tpu as pltpu
from jax.experimental.pallas import tpu_sc as plsc

_N = 8192
_D = 2048
_H = 2048
_E = 8
_NDIFF = 3
_TM = 256
_A = 2 * _N
_NT = _A // _TM + _E - 1
_APAD = _NT * _TM
_NW = 32


def _gating_body(x_ref, lab_ref, emb_ref, g1_ref, g2_ref, b_ref, idx_ref, val_ref):
    x = x_ref[...]
    logits = jnp.dot(x, g1_ref[...], preferred_element_type=jnp.float32)
    pre = jnp.dot(emb_ref[...], g2_ref[...], preferred_element_type=jnp.float32)
    lab = lab_ref[...]
    iota8 = jax.lax.broadcasted_iota(jnp.int32, (_TM, _E), 1)
    acc = jnp.zeros((_TM, _E), jnp.float32)
    for l in range(_NDIFF):
        acc = acc + jnp.where(lab == l, pre[l][None, :], 0.0)
    logits = logits + acc + b_ref[...]
    m = jnp.max(logits, axis=1, keepdims=True)
    ex = jnp.exp(logits - m)
    p = ex / jnp.sum(ex, axis=1, keepdims=True)
    m0 = jnp.max(p, axis=1, keepdims=True)
    i0 = jnp.min(jnp.where(p == m0, iota8, _E), axis=1, keepdims=True)
    sel0 = iota8 == i0
    p1 = jnp.where(sel0, -jnp.inf, p)
    m1 = jnp.max(p1, axis=1, keepdims=True)
    i1 = jnp.min(jnp.where(p1 == m1, iota8, _E), axis=1, keepdims=True)
    idx_ref[...] = jnp.where(iota8 == 0, i0, jnp.where(iota8 == 1, i1, 0))
    val_ref[...] = jnp.where(iota8 == 0, m0, jnp.where(iota8 == 1, m1, 0.0))


def _gating(x, lab, emb_pad, g1, g2, b2):
    n_tiles = _N // _TM
    return pl.pallas_call(
        _gating_body,
        grid=(n_tiles,),
        in_specs=[
            pl.BlockSpec((_TM, _D), lambda j: (j, 0)),
            pl.BlockSpec((_TM, 1), lambda j: (j, 0)),
            pl.BlockSpec((_E, _D), lambda j: (0, 0)),
            pl.BlockSpec((_D, _E), lambda j: (0, 0)),
            pl.BlockSpec((_D, _E), lambda j: (0, 0)),
            pl.BlockSpec((1, _E), lambda j: (0, 0)),
        ],
        out_specs=[
            pl.BlockSpec((_TM, _E), lambda j: (j, 0)),
            pl.BlockSpec((_TM, _E), lambda j: (j, 0)),
        ],
        out_shape=[
            jax.ShapeDtypeStruct((_N, _E), jnp.int32),
            jax.ShapeDtypeStruct((_N, _E), jnp.float32),
        ],
    )(x, lab, emb_pad, g1, g2, b2)


def _routing_meta(idx8, val8):
    ef = idx8[:, :2].reshape(-1)
    pf = val8[:, :2].reshape(-1)
    order = jnp.argsort(ef, stable=True)
    ef_sorted = ef[order]
    counts = jnp.bincount(ef, length=_E)
    csum = jnp.cumsum(counts)
    raw_start = csum - counts
    tiles_per_e = (counts + _TM - 1) // _TM
    tend = jnp.cumsum(tiles_per_e)
    tstart = tend - tiles_per_e
    pad_start = tstart * _TM
    p_idx = jnp.arange(_A, dtype=jnp.int32)
    dest = pad_start[ef_sorted] + (p_idx - raw_start[ef_sorted])
    tok_s = jnp.zeros(_APAD, jnp.int32).at[dest].set(
        (order // 2).astype(jnp.int32))
    w_s = jnp.zeros(_APAD, jnp.float32).at[dest].set(pf[order])
    pos_a = jnp.zeros(_A, jnp.int32).at[order].set(dest.astype(jnp.int32))
    p0 = pos_a[0::2]
    p1 = pos_a[1::2]
    jj = jnp.arange(_NT, dtype=jnp.int32)
    tile_e = jnp.minimum(
        jnp.sum((jj[:, None] >= tend[None, :]).astype(jnp.int32), axis=1),
        _E - 1).astype(jnp.int32)
    return tok_s, w_s, p0, p1, tile_e


_GC = 32
_GCHUNKS = _APAD // _GC
_GIT = -(-_GCHUNKS // _NW)


def _sc_gather(x, tok_s):
    mesh = plsc.VectorSubcoreMesh(core_axis_name="c", subcore_axis_name="s")

    @functools.partial(
        pl.kernel,
        mesh=mesh,
        out_type=jax.ShapeDtypeStruct((_APAD, _D), jnp.float32),
        scratch_types=[
            pltpu.VMEM((_GC,), jnp.int32),
            pltpu.VMEM((_GC, _D), jnp.float32),
            pltpu.SemaphoreType.DMA,
        ],
    )
    def k(x_hbm, tok_hbm, out_hbm, idx_v, rows_v, sem):
        wid = lax.axis_index("s") * 2 + lax.axis_index("c")

        def body(i, _):
            g = wid + i * _NW

            @pl.when(g < _GCHUNKS)
            def _():
                off = pl.multiple_of(g * _GC, _GC)
                pltpu.sync_copy(tok_hbm.at[pl.ds(off, _GC)], idx_v)
                pltpu.async_copy(x_hbm.at[idx_v], rows_v, sem).wait()
                pltpu.sync_copy(rows_v, out_hbm.at[pl.ds(off, _GC)])
            return 0

        lax.fori_loop(0, _GIT, body, 0)

    return k(x, tok_s)


def _gmm_body(te_ref, x_ref, w_ref, b_ref, wt_ref, y_ref):
    xb = x_ref[...].astype(jnp.bfloat16)
    acc = jnp.dot(xb, w_ref[0], preferred_element_type=jnp.float32)
    y_ref[...] = (acc + b_ref[0]) * wt_ref[...]


def _gmm(xs, expert_W_bf, expert_b, w_s, tile_e):
    return pl.pallas_call(
        _gmm_body,
        grid_spec=pltpu.PrefetchScalarGridSpec(
            num_scalar_prefetch=1,
            grid=(_NT,),
            in_specs=[
                pl.BlockSpec((_TM, _D), lambda j, te: (j, 0)),
                pl.BlockSpec((1, _D, _H), lambda j, te: (te[j], 0, 0)),
                pl.BlockSpec((1, 1, _H), lambda j, te: (te[j], 0, 0)),
                pl.BlockSpec((_TM, 1), lambda j, te: (j, 0)),
            ],
            out_specs=pl.BlockSpec((_TM, _H), lambda j, te: (j, 0)),
        ),
        out_shape=jax.ShapeDtypeStruct((_APAD, _H), jnp.float32),
    )(tile_e, xs, expert_W_bf, expert_b.reshape(_E, 1, _H),
      w_s.reshape(_APAD, 1))


_CC = 16
_CIT = _N // _NW // _CC


def _sc_combine(y, p0, p1):
    mesh = plsc.VectorSubcoreMesh(core_axis_name="c", subcore_axis_name="s")

    @functools.partial(
        pl.kernel,
        mesh=mesh,
        out_type=jax.ShapeDtypeStruct((_N, _H), jnp.float32),
        scratch_types=[
            pltpu.VMEM((_CC,), jnp.int32),
            pltpu.VMEM((_CC,), jnp.int32),
            pltpu.VMEM((_CC, _H), jnp.float32),
            pltpu.VMEM((_CC, _H), jnp.float32),
            pltpu.SemaphoreType.DMA,
            pltpu.SemaphoreType.DMA,
        ],
    )
    def k(y_hbm, p0_hbm, p1_hbm, out_hbm, i0_v, i1_v, y0_v, y1_v, s0, s1):
        wid = lax.axis_index("s") * 2 + lax.axis_index("c")
        base = wid * (_N // _NW)

        def body(i, _):
            off = pl.multiple_of(base + i * _CC, _CC)
            pltpu.sync_copy(p0_hbm.at[pl.ds(off, _CC)], i0_v)
            pltpu.sync_copy(p1_hbm.at[pl.ds(off, _CC)], i1_v)
            c0 = pltpu.async_copy(y_hbm.at[i0_v], y0_v, s0)
            c1 = pltpu.async_copy(y_hbm.at[i1_v], y1_v, s1)
            c0.wait()
            c1.wait()

            def radd(r, _):
                for jv in range(_H // 16):
                    sl = pl.ds(jv * 16, 16)
                    y0_v[r, sl] = y0_v[r, sl] + y1_v[r, sl]
                return 0

            lax.fori_loop(0, _CC, radd, 0)
            pltpu.sync_copy(y0_v, out_hbm.at[pl.ds(off, _CC)])
            return 0

        lax.fori_loop(0, _CIT, body, 0)

    return k(y, p0, p1)


def kernel(x, difficulty_labels, emb_table, gate_W, gate_b, expert_W, expert_b):
    lab = difficulty_labels.astype(jnp.int32).reshape(_N, 1)
    emb_pad = jnp.pad(emb_table, ((0, _E - _NDIFF), (0, 0)))
    g1 = gate_W[:_D]
    g2 = gate_W[_D:]
    b2 = gate_b.reshape(1, _E)
    idx8, val8 = _gating(x, lab, emb_pad, g1, g2, b2)
    tok_s, w_s, p0, p1, tile_e = _routing_meta(idx8, val8)
    xs = _sc_gather(x, tok_s)
    y = _gmm(xs, expert_W.astype(jnp.bfloat16), expert_b, w_s, tile_e)
    out = _sc_combine(y, p0, p1)
    return out, idx8[:, :2]


# speedup vs baseline: 1.5584x; 1.0351x over previous
import functools

import jax
import jax.numpy as jnp
from jax import lax
from jax.experimental import pallas as pl
from jax.experimental.pallas import tpu as pltpu
from jax.experimental.pallas import tpu_sc as plsc

_N = 8192
_D = 2048
_H = 2048
_E = 8
_NDIFF = 3
_TM = 256
_A = 2 * _N
_NT = _A // _TM + _E - 1
_APAD = _NT * _TM
_NW = 32


def _gating_body(x_ref, lab_ref, emb_ref, g1_ref, g2_ref, b_ref, idx_ref, val_ref):
    x = x_ref[...]
    logits = jnp.dot(x, g1_ref[...], preferred_element_type=jnp.float32)
    pre = jnp.dot(emb_ref[...], g2_ref[...], preferred_element_type=jnp.float32)
    lab = lab_ref[...]
    iota8 = jax.lax.broadcasted_iota(jnp.int32, (_TM, _E), 1)
    acc = jnp.zeros((_TM, _E), jnp.float32)
    for l in range(_NDIFF):
        acc = acc + jnp.where(lab == l, pre[l][None, :], 0.0)
    logits = logits + acc + b_ref[...]
    m = jnp.max(logits, axis=1, keepdims=True)
    ex = jnp.exp(logits - m)
    p = ex / jnp.sum(ex, axis=1, keepdims=True)
    m0 = jnp.max(p, axis=1, keepdims=True)
    i0 = jnp.min(jnp.where(p == m0, iota8, _E), axis=1, keepdims=True)
    sel0 = iota8 == i0
    p1 = jnp.where(sel0, -jnp.inf, p)
    m1 = jnp.max(p1, axis=1, keepdims=True)
    i1 = jnp.min(jnp.where(p1 == m1, iota8, _E), axis=1, keepdims=True)
    idx_ref[...] = jnp.where(iota8 == 0, i0, jnp.where(iota8 == 1, i1, 0))
    val_ref[...] = jnp.where(iota8 == 0, m0, jnp.where(iota8 == 1, m1, 0.0))


def _gating(x, lab, emb_pad, g1, g2, b2):
    n_tiles = _N // _TM
    return pl.pallas_call(
        _gating_body,
        grid=(n_tiles,),
        in_specs=[
            pl.BlockSpec((_TM, _D), lambda j: (j, 0)),
            pl.BlockSpec((_TM, 1), lambda j: (j, 0)),
            pl.BlockSpec((_E, _D), lambda j: (0, 0)),
            pl.BlockSpec((_D, _E), lambda j: (0, 0)),
            pl.BlockSpec((_D, _E), lambda j: (0, 0)),
            pl.BlockSpec((1, _E), lambda j: (0, 0)),
        ],
        out_specs=[
            pl.BlockSpec((_TM, _E), lambda j: (j, 0)),
            pl.BlockSpec((_TM, _E), lambda j: (j, 0)),
        ],
        out_shape=[
            jax.ShapeDtypeStruct((_N, _E), jnp.int32),
            jax.ShapeDtypeStruct((_N, _E), jnp.float32),
        ],
    )(x, lab, emb_pad, g1, g2, b2)


def _routing_meta(idx8, val8):
    ef = idx8[:, :2].reshape(-1)
    pf = val8[:, :2].reshape(-1)
    order = jnp.argsort(ef, stable=True)
    ef_sorted = ef[order]
    counts = jnp.bincount(ef, length=_E)
    csum = jnp.cumsum(counts)
    raw_start = csum - counts
    tiles_per_e = (counts + _TM - 1) // _TM
    tend = jnp.cumsum(tiles_per_e)
    tstart = tend - tiles_per_e
    pad_start = tstart * _TM
    p_idx = jnp.arange(_A, dtype=jnp.int32)
    dest = pad_start[ef_sorted] + (p_idx - raw_start[ef_sorted])
    tok_s = jnp.zeros(_APAD, jnp.int32).at[dest].set(
        (order // 2).astype(jnp.int32))
    w_s = jnp.zeros(_APAD, jnp.float32).at[dest].set(pf[order])
    pos_a = jnp.zeros(_A, jnp.int32).at[order].set(dest.astype(jnp.int32))
    p0 = pos_a[0::2]
    p1 = pos_a[1::2]
    jj = jnp.arange(_NT, dtype=jnp.int32)
    tile_e = jnp.minimum(
        jnp.sum((jj[:, None] >= tend[None, :]).astype(jnp.int32), axis=1),
        _E - 1).astype(jnp.int32)
    return tok_s, w_s, p0, p1, tile_e


_GPW = _APAD // _NW
_GC = 24
_GFULL = _GPW // _GC
_GTAIL = _GPW - _GFULL * _GC


def _sc_gather(x, tok_s):
    mesh = plsc.VectorSubcoreMesh(core_axis_name="c", subcore_axis_name="s")

    @functools.partial(
        pl.kernel,
        mesh=mesh,
        out_type=jax.ShapeDtypeStruct((_APAD, _D), jnp.float32),
        scratch_types=[
            pltpu.VMEM((_GPW,), jnp.int32),
            pltpu.VMEM((_GC, _D), jnp.float32),
            pltpu.VMEM((_GC, _D), jnp.float32),
            pltpu.SemaphoreType.DMA,
            pltpu.SemaphoreType.DMA,
        ],
    )
    def k(x_hbm, tok_hbm, out_hbm, idx_v, buf_a, buf_b, sem_a, sem_b):
        wid = lax.axis_index("s") * 2 + lax.axis_index("c")
        base = pl.multiple_of(wid * _GPW, _GPW)
        pltpu.sync_copy(tok_hbm.at[pl.ds(base, _GPW)], idx_v)

        def gather(c, buf, sem):
            off = pl.multiple_of(c * _GC, 8)
            pltpu.make_async_copy(x_hbm.at[idx_v.at[pl.ds(off, _GC)]],
                                  buf, sem).start()

        def gwait(buf, sem):
            pltpu.make_async_copy(x_hbm.at[idx_v.at[pl.ds(0, _GC)]],
                                  buf, sem).wait()

        def write(c, buf):
            off = pl.multiple_of(base + c * _GC, 8)
            pltpu.sync_copy(buf, out_hbm.at[pl.ds(off, _GC)])

        gather(0, buf_a, sem_a)

        def body(kk, _):
            c0 = 2 * kk
            c1 = c0 + 1
            c2 = c0 + 2

            @pl.when(c1 < _GFULL)
            def _():
                gather(c1, buf_b, sem_b)
            gwait(buf_a, sem_a)
            write(c0, buf_a)

            @pl.when(c2 < _GFULL)
            def _():
                gather(c2, buf_a, sem_a)

            @pl.when(c1 < _GFULL)
            def _():
                gwait(buf_b, sem_b)
                write(c1, buf_b)
            return 0

        lax.fori_loop(0, (_GFULL + 1) // 2, body, 0)
        toff = pl.multiple_of(_GFULL * _GC, 8)
        pltpu.async_copy(
            x_hbm.at[idx_v.at[pl.ds(toff, _GTAIL)]],
            buf_b.at[pl.ds(0, _GTAIL)], sem_b).wait()
        pltpu.sync_copy(buf_b.at[pl.ds(0, _GTAIL)],
                        out_hbm.at[pl.ds(base + _GFULL * _GC, _GTAIL)])

    return k(x, tok_s)


def _gmm_body(te_ref, x_ref, w_ref, b_ref, wt_ref, y_ref):
    xb = x_ref[...].astype(jnp.bfloat16)
    acc = jnp.dot(xb, w_ref[0], preferred_element_type=jnp.float32)
    y_ref[...] = (acc + b_ref[0]) * wt_ref[...]


def _gmm(xs, expert_W_bf, expert_b, w_s, tile_e):
    return pl.pallas_call(
        _gmm_body,
        grid_spec=pltpu.PrefetchScalarGridSpec(
            num_scalar_prefetch=1,
            grid=(_NT,),
            in_specs=[
                pl.BlockSpec((_TM, _D), lambda j, te: (j, 0)),
                pl.BlockSpec((1, _D, _H), lambda j, te: (te[j], 0, 0)),
                pl.BlockSpec((1, 1, _H), lambda j, te: (te[j], 0, 0)),
                pl.BlockSpec((_TM, 1), lambda j, te: (j, 0)),
            ],
            out_specs=pl.BlockSpec((_TM, _H), lambda j, te: (j, 0)),
        ),
        out_shape=jax.ShapeDtypeStruct((_APAD, _H), jnp.float32),
    )(tile_e, xs, expert_W_bf, expert_b.reshape(_E, 1, _H),
      w_s.reshape(_APAD, 1))


_CPW = _N // _NW
_CC = 8
_CCHUNKS = _CPW // _CC


def _sc_combine(y, p0, p1):
    mesh = plsc.VectorSubcoreMesh(core_axis_name="c", subcore_axis_name="s")

    @functools.partial(
        pl.kernel,
        mesh=mesh,
        out_type=jax.ShapeDtypeStruct((_N, _H), jnp.float32),
        scratch_types=[
            pltpu.VMEM((_CPW,), jnp.int32),
            pltpu.VMEM((_CPW,), jnp.int32),
            pltpu.VMEM((_CC, _H), jnp.float32),
            pltpu.VMEM((_CC, _H), jnp.float32),
            pltpu.VMEM((_CC, _H), jnp.float32),
            pltpu.VMEM((_CC, _H), jnp.float32),
            pltpu.SemaphoreType.DMA,
            pltpu.SemaphoreType.DMA,
        ],
    )
    def k(y_hbm, p0_hbm, p1_hbm, out_hbm, i0_v, i1_v,
          a0_v, a1_v, b0_v, b1_v, sa, sb):
        wid = lax.axis_index("s") * 2 + lax.axis_index("c")
        base = pl.multiple_of(wid * _CPW, _CPW)
        pltpu.sync_copy(p0_hbm.at[pl.ds(base, _CPW)], i0_v)
        pltpu.sync_copy(p1_hbm.at[pl.ds(base, _CPW)], i1_v)

        def gather(c, r0, r1, sem):
            off = pl.multiple_of(c * _CC, 8)
            pltpu.make_async_copy(y_hbm.at[i0_v.at[pl.ds(off, _CC)]],
                                  r0, sem).start()
            pltpu.make_async_copy(y_hbm.at[i1_v.at[pl.ds(off, _CC)]],
                                  r1, sem).start()

        def gwait(r0, r1, sem):
            pltpu.make_async_copy(y_hbm.at[i0_v.at[pl.ds(0, _CC)]],
                                  r0, sem).wait()
            pltpu.make_async_copy(y_hbm.at[i1_v.at[pl.ds(0, _CC)]],
                                  r1, sem).wait()

        def addwrite(c, r0, r1):
            def radd(r, _):
                for jv in range(_H // 16):
                    sl = pl.ds(jv * 16, 16)
                    r0[r, sl] = r0[r, sl] + r1[r, sl]
                return 0

            lax.fori_loop(0, _CC, radd, 0)
            pltpu.sync_copy(r0, out_hbm.at[pl.ds(base + c * _CC, _CC)])

        gather(0, a0_v, a1_v, sa)

        def body(kk, _):
            c0 = 2 * kk
            c1 = c0 + 1
            c2 = c0 + 2
            gather(c1, b0_v, b1_v, sb)
            gwait(a0_v, a1_v, sa)
            addwrite(c0, a0_v, a1_v)

            @pl.when(c2 < _CCHUNKS)
            def _():
                gather(c2, a0_v, a1_v, sa)
            gwait(b0_v, b1_v, sb)
            addwrite(c1, b0_v, b1_v)
            return 0

        lax.fori_loop(0, _CCHUNKS // 2, body, 0)

    return k(y, p0, p1)


def kernel(x, difficulty_labels, emb_table, gate_W, gate_b, expert_W, expert_b):
    lab = difficulty_labels.astype(jnp.int32).reshape(_N, 1)
    emb_pad = jnp.pad(emb_table, ((0, _E - _NDIFF), (0, 0)))
    g1 = gate_W[:_D]
    g2 = gate_W[_D:]
    b2 = gate_b.reshape(1, _E)
    idx8, val8 = _gating(x, lab, emb_pad, g1, g2, b2)
    tok_s, w_s, p0, p1, tile_e = _routing_meta(idx8, val8)
    xs = _sc_gather(x, tok_s)
    y = _gmm(xs, expert_W.astype(jnp.bfloat16), expert_b, w_s, tile_e)
    out = _sc_combine(y, p0, p1)
    return out, idx8[:, :2]


# speedup vs baseline: 1.6113x; 1.0339x over previous
import functools

import jax
import jax.numpy as jnp
from jax import lax
from jax.experimental import pallas as pl
from jax.experimental.pallas import tpu as pltpu
from jax.experimental.pallas import tpu_sc as plsc

_N = 8192
_D = 2048
_H = 2048
_E = 8
_NDIFF = 3
_TM = 256
_A = 2 * _N
_NT = _A // _TM + _E - 1
_APAD = _NT * _TM
_NW = 32


def _gating_body(x_ref, lab_ref, emb_ref, g1_ref, g2_ref, b_ref,
                 idx_ref, val_ref, xbf_ref):
    x = x_ref[...]
    xb = x.astype(jnp.bfloat16)
    lo = jax.lax.bitcast_convert_type(xb[:, :_D // 2], jnp.uint16).astype(jnp.int32)
    hi = jax.lax.bitcast_convert_type(xb[:, _D // 2:], jnp.uint16).astype(jnp.int32)
    xbf_ref[...] = lo | (hi << 16)
    logits = jnp.dot(x, g1_ref[...], preferred_element_type=jnp.float32)
    pre = jnp.dot(emb_ref[...], g2_ref[...], preferred_element_type=jnp.float32)
    lab = lab_ref[...]
    iota8 = jax.lax.broadcasted_iota(jnp.int32, (_TM, _E), 1)
    acc = jnp.zeros((_TM, _E), jnp.float32)
    for l in range(_NDIFF):
        acc = acc + jnp.where(lab == l, pre[l][None, :], 0.0)
    logits = logits + acc + b_ref[...]
    m = jnp.max(logits, axis=1, keepdims=True)
    ex = jnp.exp(logits - m)
    p = ex / jnp.sum(ex, axis=1, keepdims=True)
    m0 = jnp.max(p, axis=1, keepdims=True)
    i0 = jnp.min(jnp.where(p == m0, iota8, _E), axis=1, keepdims=True)
    sel0 = iota8 == i0
    p1 = jnp.where(sel0, -jnp.inf, p)
    m1 = jnp.max(p1, axis=1, keepdims=True)
    i1 = jnp.min(jnp.where(p1 == m1, iota8, _E), axis=1, keepdims=True)
    idx_ref[...] = jnp.where(iota8 == 0, i0, jnp.where(iota8 == 1, i1, 0))
    val_ref[...] = jnp.where(iota8 == 0, m0, jnp.where(iota8 == 1, m1, 0.0))


def _gating(x, lab, emb_pad, g1, g2, b2):
    n_tiles = _N // _TM
    return pl.pallas_call(
        _gating_body,
        grid=(n_tiles,),
        in_specs=[
            pl.BlockSpec((_TM, _D), lambda j: (j, 0)),
            pl.BlockSpec((_TM, 1), lambda j: (j, 0)),
            pl.BlockSpec((_E, _D), lambda j: (0, 0)),
            pl.BlockSpec((_D, _E), lambda j: (0, 0)),
            pl.BlockSpec((_D, _E), lambda j: (0, 0)),
            pl.BlockSpec((1, _E), lambda j: (0, 0)),
        ],
        out_specs=[
            pl.BlockSpec((_TM, _E), lambda j: (j, 0)),
            pl.BlockSpec((_TM, _E), lambda j: (j, 0)),
            pl.BlockSpec((_TM, _D // 2), lambda j: (j, 0)),
        ],
        out_shape=[
            jax.ShapeDtypeStruct((_N, _E), jnp.int32),
            jax.ShapeDtypeStruct((_N, _E), jnp.float32),
            jax.ShapeDtypeStruct((_N, _D // 2), jnp.int32),
        ],
    )(x, lab, emb_pad, g1, g2, b2)


def _routing_meta(idx8, val8):
    ef = idx8[:, :2].reshape(-1)
    pf = val8[:, :2].reshape(-1)
    order = jnp.argsort(ef, stable=True)
    ef_sorted = ef[order]
    counts = jnp.bincount(ef, length=_E)
    csum = jnp.cumsum(counts)
    raw_start = csum - counts
    tiles_per_e = (counts + _TM - 1) // _TM
    tend = jnp.cumsum(tiles_per_e)
    tstart = tend - tiles_per_e
    pad_start = tstart * _TM
    p_idx = jnp.arange(_A, dtype=jnp.int32)
    dest = pad_start[ef_sorted] + (p_idx - raw_start[ef_sorted])
    tok_s = jnp.zeros(_APAD, jnp.int32).at[dest].set(
        (order // 2).astype(jnp.int32))
    w_s = jnp.zeros(_APAD, jnp.float32).at[dest].set(pf[order])
    pos_a = jnp.zeros(_A, jnp.int32).at[order].set(dest.astype(jnp.int32))
    p0 = pos_a[0::2]
    p1 = pos_a[1::2]
    jj = jnp.arange(_NT, dtype=jnp.int32)
    tile_e = jnp.minimum(
        jnp.sum((jj[:, None] >= tend[None, :]).astype(jnp.int32), axis=1),
        _E - 1).astype(jnp.int32)
    return tok_s, w_s, p0, p1, tile_e


_GPW = _APAD // _NW
_GC = 48
_GFULL = _GPW // _GC
_GTAIL = _GPW - _GFULL * _GC


def _sc_gather(x, tok_s):
    mesh = plsc.VectorSubcoreMesh(core_axis_name="c", subcore_axis_name="s")

    @functools.partial(
        pl.kernel,
        mesh=mesh,
        out_type=jax.ShapeDtypeStruct((_APAD, _D // 2), jnp.int32),
        scratch_types=[
            pltpu.VMEM((_GPW,), jnp.int32),
            pltpu.VMEM((_GC, _D // 2), jnp.int32),
            pltpu.VMEM((_GC, _D // 2), jnp.int32),
            pltpu.SemaphoreType.DMA,
            pltpu.SemaphoreType.DMA,
        ],
    )
    def k(x_hbm, tok_hbm, out_hbm, idx_v, buf_a, buf_b, sem_a, sem_b):
        wid = lax.axis_index("s") * 2 + lax.axis_index("c")
        base = pl.multiple_of(wid * _GPW, _GPW)
        pltpu.sync_copy(tok_hbm.at[pl.ds(base, _GPW)], idx_v)

        def gather(c, buf, sem):
            off = pl.multiple_of(c * _GC, 8)
            pltpu.make_async_copy(x_hbm.at[idx_v.at[pl.ds(off, _GC)]],
                                  buf, sem).start()

        def gwait(buf, sem):
            pltpu.make_async_copy(x_hbm.at[idx_v.at[pl.ds(0, _GC)]],
                                  buf, sem).wait()

        def write(c, buf):
            off = pl.multiple_of(base + c * _GC, 8)
            pltpu.sync_copy(buf, out_hbm.at[pl.ds(off, _GC)])

        gather(0, buf_a, sem_a)

        def body(kk, _):
            c0 = 2 * kk
            c1 = c0 + 1
            c2 = c0 + 2

            @pl.when(c1 < _GFULL)
            def _():
                gather(c1, buf_b, sem_b)
            gwait(buf_a, sem_a)
            write(c0, buf_a)

            @pl.when(c2 < _GFULL)
            def _():
                gather(c2, buf_a, sem_a)

            @pl.when(c1 < _GFULL)
            def _():
                gwait(buf_b, sem_b)
                write(c1, buf_b)
            return 0

        lax.fori_loop(0, (_GFULL + 1) // 2, body, 0)
        toff = pl.multiple_of(_GFULL * _GC, 8)
        pltpu.async_copy(
            x_hbm.at[idx_v.at[pl.ds(toff, _GTAIL)]],
            buf_b.at[pl.ds(0, _GTAIL)], sem_b).wait()
        pltpu.sync_copy(buf_b.at[pl.ds(0, _GTAIL)],
                        out_hbm.at[pl.ds(base + _GFULL * _GC, _GTAIL)])

    return k(x, tok_s)


def _gmm_body(te_ref, x_ref, w_ref, b_ref, wt_ref, y_ref):
    xp = x_ref[...]
    x_lo = jax.lax.bitcast_convert_type(
        xp << 16, jnp.float32).astype(jnp.bfloat16)
    x_hi = jax.lax.bitcast_convert_type(
        xp & jnp.int32(-65536), jnp.float32).astype(jnp.bfloat16)
    acc = jnp.dot(x_lo, w_ref[0, :_D // 2, :],
                  preferred_element_type=jnp.float32)
    acc = acc + jnp.dot(x_hi, w_ref[0, _D // 2:, :],
                        preferred_element_type=jnp.float32)
    y_ref[...] = (acc + b_ref[0]) * wt_ref[...]


def _gmm(xs, expert_W_bf, expert_b, w_s, tile_e):
    return pl.pallas_call(
        _gmm_body,
        grid_spec=pltpu.PrefetchScalarGridSpec(
            num_scalar_prefetch=1,
            grid=(_NT,),
            in_specs=[
                pl.BlockSpec((_TM, _D // 2), lambda j, te: (j, 0)),
                pl.BlockSpec((1, _D, _H), lambda j, te: (te[j], 0, 0)),
                pl.BlockSpec((1, 1, _H), lambda j, te: (te[j], 0, 0)),
                pl.BlockSpec((_TM, 1), lambda j, te: (j, 0)),
            ],
            out_specs=pl.BlockSpec((_TM, _H), lambda j, te: (j, 0)),
        ),
        out_shape=jax.ShapeDtypeStruct((_APAD, _H), jnp.float32),
    )(tile_e, xs, expert_W_bf, expert_b.reshape(_E, 1, _H),
      w_s.reshape(_APAD, 1))


_CPW = _N // _NW
_CC = 8
_CCHUNKS = _CPW // _CC


def _sc_combine(y, p0, p1):
    mesh = plsc.VectorSubcoreMesh(core_axis_name="c", subcore_axis_name="s")

    @functools.partial(
        pl.kernel,
        mesh=mesh,
        out_type=jax.ShapeDtypeStruct((_N, _H), jnp.float32),
        scratch_types=[
            pltpu.VMEM((_CPW,), jnp.int32),
            pltpu.VMEM((_CPW,), jnp.int32),
            pltpu.VMEM((_CC, _H), jnp.float32),
            pltpu.VMEM((_CC, _H), jnp.float32),
            pltpu.VMEM((_CC, _H), jnp.float32),
            pltpu.VMEM((_CC, _H), jnp.float32),
            pltpu.SemaphoreType.DMA,
            pltpu.SemaphoreType.DMA,
        ],
    )
    def k(y_hbm, p0_hbm, p1_hbm, out_hbm, i0_v, i1_v,
          a0_v, a1_v, b0_v, b1_v, sa, sb):
        wid = lax.axis_index("s") * 2 + lax.axis_index("c")
        base = pl.multiple_of(wid * _CPW, _CPW)
        pltpu.sync_copy(p0_hbm.at[pl.ds(base, _CPW)], i0_v)
        pltpu.sync_copy(p1_hbm.at[pl.ds(base, _CPW)], i1_v)

        def gather(c, r0, r1, sem):
            off = pl.multiple_of(c * _CC, 8)
            pltpu.make_async_copy(y_hbm.at[i0_v.at[pl.ds(off, _CC)]],
                                  r0, sem).start()
            pltpu.make_async_copy(y_hbm.at[i1_v.at[pl.ds(off, _CC)]],
                                  r1, sem).start()

        def gwait(r0, r1, sem):
            pltpu.make_async_copy(y_hbm.at[i0_v.at[pl.ds(0, _CC)]],
                                  r0, sem).wait()
            pltpu.make_async_copy(y_hbm.at[i1_v.at[pl.ds(0, _CC)]],
                                  r1, sem).wait()

        def addwrite(c, r0, r1):
            def radd(r, _):
                for jv in range(_H // 16):
                    sl = pl.ds(jv * 16, 16)
                    r0[r, sl] = r0[r, sl] + r1[r, sl]
                return 0

            lax.fori_loop(0, _CC, radd, 0)
            pltpu.sync_copy(r0, out_hbm.at[pl.ds(base + c * _CC, _CC)])

        gather(0, a0_v, a1_v, sa)

        def body(kk, _):
            c0 = 2 * kk
            c1 = c0 + 1
            c2 = c0 + 2
            gather(c1, b0_v, b1_v, sb)
            gwait(a0_v, a1_v, sa)
            addwrite(c0, a0_v, a1_v)

            @pl.when(c2 < _CCHUNKS)
            def _():
                gather(c2, a0_v, a1_v, sa)
            gwait(b0_v, b1_v, sb)
            addwrite(c1, b0_v, b1_v)
            return 0

        lax.fori_loop(0, _CCHUNKS // 2, body, 0)

    return k(y, p0, p1)


def kernel(x, difficulty_labels, emb_table, gate_W, gate_b, expert_W, expert_b):
    lab = difficulty_labels.astype(jnp.int32).reshape(_N, 1)
    emb_pad = jnp.pad(emb_table, ((0, _E - _NDIFF), (0, 0)))
    g1 = gate_W[:_D]
    g2 = gate_W[_D:]
    b2 = gate_b.reshape(1, _E)
    idx8, val8, xbf = _gating(x, lab, emb_pad, g1, g2, b2)
    tok_s, w_s, p0, p1, tile_e = _routing_meta(idx8, val8)
    xs = _sc_gather(xbf, tok_s)
    y = _gmm(xs, expert_W.astype(jnp.bfloat16), expert_b, w_s, tile_e)
    out = _sc_combine(y, p0, p1)
    return out, idx8[:, :2]


# speedup vs baseline: 1.6926x; 1.0504x over previous
import functools

import jax
import jax.numpy as jnp
from jax import lax
from jax.experimental import pallas as pl
from jax.experimental.pallas import tpu as pltpu
from jax.experimental.pallas import tpu_sc as plsc

_N = 8192
_D = 2048
_H = 2048
_E = 8
_NDIFF = 3
_TM = 256
_A = 2 * _N
_NT = _A // _TM + _E - 1
_APAD = _NT * _TM
_NW = 32


def _gating_body(x_ref, lab_ref, emb_ref, g1_ref, g2_ref, b_ref,
                 idx_ref, val_ref, xbf_ref):
    x = x_ref[...]
    xb = x.astype(jnp.bfloat16)
    lo = jax.lax.bitcast_convert_type(xb[:, :_D // 2], jnp.uint16).astype(jnp.int32)
    hi = jax.lax.bitcast_convert_type(xb[:, _D // 2:], jnp.uint16).astype(jnp.int32)
    xbf_ref[...] = lo | (hi << 16)
    logits = jnp.dot(x, g1_ref[...], preferred_element_type=jnp.float32)
    pre = jnp.dot(emb_ref[...], g2_ref[...], preferred_element_type=jnp.float32)
    lab = lab_ref[...]
    iota8 = jax.lax.broadcasted_iota(jnp.int32, (_TM, _E), 1)
    acc = jnp.zeros((_TM, _E), jnp.float32)
    for l in range(_NDIFF):
        acc = acc + jnp.where(lab == l, pre[l][None, :], 0.0)
    logits = logits + acc + b_ref[...]
    m = jnp.max(logits, axis=1, keepdims=True)
    ex = jnp.exp(logits - m)
    p = ex / jnp.sum(ex, axis=1, keepdims=True)
    m0 = jnp.max(p, axis=1, keepdims=True)
    i0 = jnp.min(jnp.where(p == m0, iota8, _E), axis=1, keepdims=True)
    sel0 = iota8 == i0
    p1 = jnp.where(sel0, -jnp.inf, p)
    m1 = jnp.max(p1, axis=1, keepdims=True)
    i1 = jnp.min(jnp.where(p1 == m1, iota8, _E), axis=1, keepdims=True)
    idx_ref[...] = jnp.where(iota8 == 0, i0, jnp.where(iota8 == 1, i1, 0))
    val_ref[...] = jnp.where(iota8 == 0, m0, jnp.where(iota8 == 1, m1, 0.0))


def _gating(x, lab, emb_pad, g1, g2, b2):
    n_tiles = _N // _TM
    return pl.pallas_call(
        _gating_body,
        grid=(n_tiles,),
        in_specs=[
            pl.BlockSpec((_TM, _D), lambda j: (j, 0)),
            pl.BlockSpec((_TM, 1), lambda j: (j, 0)),
            pl.BlockSpec((_E, _D), lambda j: (0, 0)),
            pl.BlockSpec((_D, _E), lambda j: (0, 0)),
            pl.BlockSpec((_D, _E), lambda j: (0, 0)),
            pl.BlockSpec((1, _E), lambda j: (0, 0)),
        ],
        out_specs=[
            pl.BlockSpec((_TM, _E), lambda j: (j, 0)),
            pl.BlockSpec((_TM, _E), lambda j: (j, 0)),
            pl.BlockSpec((_TM, _D // 2), lambda j: (j, 0)),
        ],
        out_shape=[
            jax.ShapeDtypeStruct((_N, _E), jnp.int32),
            jax.ShapeDtypeStruct((_N, _E), jnp.float32),
            jax.ShapeDtypeStruct((_N, _D // 2), jnp.int32),
        ],
    )(x, lab, emb_pad, g1, g2, b2)


def _routing_meta(idx8, val8):
    ef = idx8[:, :2].reshape(-1)
    pf = val8[:, :2].reshape(-1)
    onehot = (ef[:, None] == jnp.arange(_E, dtype=jnp.int32)[None, :])
    csum = jnp.cumsum(onehot.astype(jnp.int32), axis=0)
    counts = csum[-1]
    rank = jnp.take_along_axis(csum, ef[:, None], axis=1)[:, 0] - 1
    tiles_per_e = (counts + _TM - 1) // _TM
    tend = jnp.cumsum(tiles_per_e)
    pad_start = (tend - tiles_per_e) * _TM
    dest = (pad_start[ef] + rank).astype(jnp.int32)
    a_idx = jnp.arange(_A, dtype=jnp.int32)
    tok_s = jnp.zeros(_APAD, jnp.int32).at[dest].set(a_idx // 2)
    w_s = jnp.zeros(_APAD, jnp.float32).at[dest].set(pf)
    p0 = dest[0::2]
    p1 = dest[1::2]
    jj = jnp.arange(_NT, dtype=jnp.int32)
    tile_e = jnp.minimum(
        jnp.sum((jj[:, None] >= tend[None, :]).astype(jnp.int32), axis=1),
        _E - 1).astype(jnp.int32)
    return tok_s, w_s, p0, p1, tile_e


_GPW = _APAD // _NW
_GC = 48
_GFULL = _GPW // _GC
_GTAIL = _GPW - _GFULL * _GC


def _sc_gather(x, tok_s):
    mesh = plsc.VectorSubcoreMesh(core_axis_name="c", subcore_axis_name="s")

    @functools.partial(
        pl.kernel,
        mesh=mesh,
        out_type=jax.ShapeDtypeStruct((_APAD, _D // 2), jnp.int32),
        scratch_types=[
            pltpu.VMEM((_GPW,), jnp.int32),
            pltpu.VMEM((_GC, _D // 2), jnp.int32),
            pltpu.VMEM((_GC, _D // 2), jnp.int32),
            pltpu.SemaphoreType.DMA,
            pltpu.SemaphoreType.DMA,
        ],
    )
    def k(x_hbm, tok_hbm, out_hbm, idx_v, buf_a, buf_b, sem_a, sem_b):
        wid = lax.axis_index("s") * 2 + lax.axis_index("c")
        base = pl.multiple_of(wid * _GPW, _GPW)
        pltpu.sync_copy(tok_hbm.at[pl.ds(base, _GPW)], idx_v)

        def gather(c, buf, sem):
            off = pl.multiple_of(c * _GC, 8)
            pltpu.make_async_copy(x_hbm.at[idx_v.at[pl.ds(off, _GC)]],
                                  buf, sem).start()

        def gwait(buf, sem):
            pltpu.make_async_copy(x_hbm.at[idx_v.at[pl.ds(0, _GC)]],
                                  buf, sem).wait()

        def write(c, buf):
            off = pl.multiple_of(base + c * _GC, 8)
            pltpu.sync_copy(buf, out_hbm.at[pl.ds(off, _GC)])

        gather(0, buf_a, sem_a)

        def body(kk, _):
            c0 = 2 * kk
            c1 = c0 + 1
            c2 = c0 + 2

            @pl.when(c1 < _GFULL)
            def _():
                gather(c1, buf_b, sem_b)
            gwait(buf_a, sem_a)
            write(c0, buf_a)

            @pl.when(c2 < _GFULL)
            def _():
                gather(c2, buf_a, sem_a)

            @pl.when(c1 < _GFULL)
            def _():
                gwait(buf_b, sem_b)
                write(c1, buf_b)
            return 0

        lax.fori_loop(0, (_GFULL + 1) // 2, body, 0)
        toff = pl.multiple_of(_GFULL * _GC, 8)
        pltpu.async_copy(
            x_hbm.at[idx_v.at[pl.ds(toff, _GTAIL)]],
            buf_b.at[pl.ds(0, _GTAIL)], sem_b).wait()
        pltpu.sync_copy(buf_b.at[pl.ds(0, _GTAIL)],
                        out_hbm.at[pl.ds(base + _GFULL * _GC, _GTAIL)])

    return k(x, tok_s)


def _gmm_body(te_ref, x_ref, w_ref, b_ref, wt_ref, y_ref):
    xp = x_ref[...]
    x_lo = jax.lax.bitcast_convert_type(
        xp << 16, jnp.float32).astype(jnp.bfloat16)
    x_hi = jax.lax.bitcast_convert_type(
        xp & jnp.int32(-65536), jnp.float32).astype(jnp.bfloat16)
    acc = jnp.dot(x_lo, w_ref[0, :_D // 2, :],
                  preferred_element_type=jnp.float32)
    acc = acc + jnp.dot(x_hi, w_ref[0, _D // 2:, :],
                        preferred_element_type=jnp.float32)
    y_ref[...] = (acc + b_ref[0]) * wt_ref[...]


def _gmm(xs, expert_W_bf, expert_b, w_s, tile_e):
    return pl.pallas_call(
        _gmm_body,
        grid_spec=pltpu.PrefetchScalarGridSpec(
            num_scalar_prefetch=1,
            grid=(_NT,),
            in_specs=[
                pl.BlockSpec((_TM, _D // 2), lambda j, te: (j, 0)),
                pl.BlockSpec((1, _D, _H), lambda j, te: (te[j], 0, 0)),
                pl.BlockSpec((1, 1, _H), lambda j, te: (te[j], 0, 0)),
                pl.BlockSpec((_TM, 1), lambda j, te: (j, 0)),
            ],
            out_specs=pl.BlockSpec((_TM, _H), lambda j, te: (j, 0)),
        ),
        out_shape=jax.ShapeDtypeStruct((_APAD, _H), jnp.float32),
    )(tile_e, xs, expert_W_bf, expert_b.reshape(_E, 1, _H),
      w_s.reshape(_APAD, 1))


_CPW = _N // _NW
_CC = 8
_CCHUNKS = _CPW // _CC


def _sc_combine(y, p0, p1):
    mesh = plsc.VectorSubcoreMesh(core_axis_name="c", subcore_axis_name="s")

    @functools.partial(
        pl.kernel,
        mesh=mesh,
        out_type=jax.ShapeDtypeStruct((_N, _H), jnp.float32),
        scratch_types=[
            pltpu.VMEM((_CPW,), jnp.int32),
            pltpu.VMEM((_CPW,), jnp.int32),
            pltpu.VMEM((_CC, _H), jnp.float32),
            pltpu.VMEM((_CC, _H), jnp.float32),
            pltpu.VMEM((_CC, _H), jnp.float32),
            pltpu.VMEM((_CC, _H), jnp.float32),
            pltpu.SemaphoreType.DMA,
            pltpu.SemaphoreType.DMA,
        ],
    )
    def k(y_hbm, p0_hbm, p1_hbm, out_hbm, i0_v, i1_v,
          a0_v, a1_v, b0_v, b1_v, sa, sb):
        wid = lax.axis_index("s") * 2 + lax.axis_index("c")
        base = pl.multiple_of(wid * _CPW, _CPW)
        pltpu.sync_copy(p0_hbm.at[pl.ds(base, _CPW)], i0_v)
        pltpu.sync_copy(p1_hbm.at[pl.ds(base, _CPW)], i1_v)

        def gather(c, r0, r1, sem):
            off = pl.multiple_of(c * _CC, 8)
            pltpu.make_async_copy(y_hbm.at[i0_v.at[pl.ds(off, _CC)]],
                                  r0, sem).start()
            pltpu.make_async_copy(y_hbm.at[i1_v.at[pl.ds(off, _CC)]],
                                  r1, sem).start()

        def gwait(r0, r1, sem):
            pltpu.make_async_copy(y_hbm.at[i0_v.at[pl.ds(0, _CC)]],
                                  r0, sem).wait()
            pltpu.make_async_copy(y_hbm.at[i1_v.at[pl.ds(0, _CC)]],
                                  r1, sem).wait()

        def addwrite(c, r0, r1):
            def radd(r, _):
                for jv in range(_H // 16):
                    sl = pl.ds(jv * 16, 16)
                    r0[r, sl] = r0[r, sl] + r1[r, sl]
                return 0

            lax.fori_loop(0, _CC, radd, 0)
            pltpu.sync_copy(r0, out_hbm.at[pl.ds(base + c * _CC, _CC)])

        gather(0, a0_v, a1_v, sa)

        def body(kk, _):
            c0 = 2 * kk
            c1 = c0 + 1
            c2 = c0 + 2
            gather(c1, b0_v, b1_v, sb)
            gwait(a0_v, a1_v, sa)
            addwrite(c0, a0_v, a1_v)

            @pl.when(c2 < _CCHUNKS)
            def _():
                gather(c2, a0_v, a1_v, sa)
            gwait(b0_v, b1_v, sb)
            addwrite(c1, b0_v, b1_v)
            return 0

        lax.fori_loop(0, _CCHUNKS // 2, body, 0)

    return k(y, p0, p1)


def kernel(x, difficulty_labels, emb_table, gate_W, gate_b, expert_W, expert_b):
    lab = difficulty_labels.astype(jnp.int32).reshape(_N, 1)
    emb_pad = jnp.pad(emb_table, ((0, _E - _NDIFF), (0, 0)))
    g1 = gate_W[:_D]
    g2 = gate_W[_D:]
    b2 = gate_b.reshape(1, _E)
    idx8, val8, xbf = _gating(x, lab, emb_pad, g1, g2, b2)
    tok_s, w_s, p0, p1, tile_e = _routing_meta(idx8, val8)
    xs = _sc_gather(xbf, tok_s)
    y = _gmm(xs, expert_W.astype(jnp.bfloat16), expert_b, w_s, tile_e)
    out = _sc_combine(y, p0, p1)
    return out, idx8[:, :2]
